# baseline (speedup 1.0000x reference)
import functools

import jax
import jax.numpy as jnp
from jax import lax
from jax.experimental import pallas as pl
from jax.experimental.pallas import tpu as pltpu
from jax.experimental.pallas import tpu_sc as plsc

_N, _D, _H, _G, _M = 10000, 128, 64, 128, 32
_NC, _NS = 2, 16
_NW = _NC * _NS
_NP = 10240
_CHUNK = 128
_CPW = 80
_CP0 = 40
_CP1 = 2 * _CPW - _CP0
_CPX = max(_CP0, _CP1)
_EP = _NW * _CPW * _CHUNK
_RPT = _NP // _NS
_F32 = jnp.float32


def _sc_mesh():
    return plsc.VectorSubcoreMesh(
        core_axis_name="c", subcore_axis_name="s",
        num_cores=_NC, num_subcores=_NS)


def _sc_degree(dst2d, ones16, zero16):
    @functools.partial(
        pl.kernel,
        out_type=jax.ShapeDtypeStruct((_NC * _NP, 16), _F32),
        mesh=_sc_mesh(),
        scratch_types=[
            pltpu.VMEM((_CPW, _CHUNK), jnp.int32),
            pltpu.VMEM((_CHUNK, 16), _F32),
            pltpu.VMEM((_RPT, 16), _F32),
            pltpu.VMEM_SHARED((_NP, 16), _F32),
            pltpu.SemaphoreType.DMA,
        ],
        compiler_params=pltpu.CompilerParams(use_tc_tiling_on_sc=False),
    )
    def body(dst_hbm, ones_hbm, zero_hbm, out_hbm, dst_v, ones_v, tmp_v,
             deg_sh, sem):
        cid = lax.axis_index("c")
        sid = lax.axis_index("s")
        w = cid * _NS + sid
        pltpu.sync_copy(zero_hbm, ones_v)
        for k in range(_RPT // _CHUNK):
            pltpu.sync_copy(ones_v, deg_sh.at[pl.ds(sid * _RPT + k * _CHUNK,
                                                    _CHUNK)])
        pltpu.sync_copy(ones_hbm, ones_v)
        pltpu.sync_copy(dst_hbm.at[pl.ds(w * _CPW, _CPW)], dst_v)
        plsc.subcore_barrier()

        def step(j, carry):
            pltpu.sync_copy(ones_v, deg_sh.at[dst_v.at[j]], add=True)
            return carry
        lax.fori_loop(0, _CPW, step, 0)
        plsc.subcore_barrier()

        pltpu.sync_copy(deg_sh.at[pl.ds(sid * _RPT, _RPT)], tmp_v)
        pltpu.sync_copy(tmp_v, out_hbm.at[pl.ds(cid * _NP + sid * _RPT, _RPT)])

    return body(dst2d, ones16, zero16)


def _sc_edge_pass(y, src2d, dst2d, zero64):
    @functools.partial(
        pl.kernel,
        out_type=jax.ShapeDtypeStruct((_NC * _NP, _H), _F32),
        mesh=_sc_mesh(),
        scratch_types=[
            pltpu.VMEM((_CPX, _CHUNK), jnp.int32),
            pltpu.VMEM((_CPX, _CHUNK), jnp.int32),
            pltpu.VMEM((_CHUNK, _H), _F32),
            pltpu.VMEM((_CHUNK, _H), _F32),
            pltpu.VMEM((_CHUNK, _H), _F32),
            pltpu.VMEM_SHARED((_NP, _H), _F32),
            pltpu.SemaphoreType.DMA,
            pltpu.SemaphoreType.DMA,
        ],
        compiler_params=pltpu.CompilerParams(use_tc_tiling_on_sc=False),
    )
    def body(y_hbm, src_hbm, dst_hbm, zero_hbm, out_hbm, src_v, dst_v,
             rows_a, rows_b, tmp_v, acc_sh, sem_a, sem_b):
        cid = lax.axis_index("c")
        sid = lax.axis_index("s")
        pltpu.sync_copy(zero_hbm, tmp_v)
        for k in range(_RPT // _CHUNK):
            pltpu.sync_copy(tmp_v, acc_sh.at[pl.ds(sid * _RPT + k * _CHUNK,
                                                   _CHUNK)])
        n = jnp.where(cid == 0, _CP0, _CP1)
        base = cid * (_NS * _CP0) + sid * n
        pltpu.sync_copy(src_hbm.at[pl.ds(base, _CPX)], src_v)
        pltpu.sync_copy(dst_hbm.at[pl.ds(base, _CPX)], dst_v)
        plsc.subcore_barrier()

        pltpu.async_copy(y_hbm.at[src_v.at[0]], rows_a, sem_a)
        half = n // 2

        def step(i, carry):
            j0 = 2 * i
            j1 = 2 * i + 1
            pltpu.make_async_copy(y_hbm.at[src_v.at[j0]], rows_a,
                                  sem_a).wait()
            pltpu.async_copy(y_hbm.at[src_v.at[j1]], rows_b, sem_b)
            pltpu.sync_copy(rows_a, acc_sh.at[dst_v.at[j0]], add=True)

            @pl.when(i < half - 1)
            def _():
                pltpu.async_copy(y_hbm.at[src_v.at[j0 + 2]], rows_a, sem_a)
            pltpu.make_async_copy(y_hbm.at[src_v.at[j1]], rows_b,
                                  sem_b).wait()
            pltpu.sync_copy(rows_b, acc_sh.at[dst_v.at[j1]], add=True)
            return carry
        lax.fori_loop(0, half, step, 0)
        plsc.subcore_barrier()

        for k in range(_RPT // _CHUNK):
            r0 = sid * _RPT + k * _CHUNK
            pltpu.sync_copy(acc_sh.at[pl.ds(r0, _CHUNK)], tmp_v)
            pltpu.sync_copy(tmp_v, out_hbm.at[pl.ds(cid * _NP + r0, _CHUNK)])

    return body(y, src2d, dst2d, zero64)


_BINS = 136
_RPW = _NP // _NW


def _sc_pool(acc2, y, dis16, b_row, batch_pad):
    @functools.partial(
        pl.kernel,
        out_type=(jax.ShapeDtypeStruct((_NW * _BINS, _H), _F32),
                  jax.ShapeDtypeStruct((_NW * _BINS, _H), _F32),
                  jax.ShapeDtypeStruct((_NW * _BINS, 16), _F32)),
        mesh=_sc_mesh(),
        scratch_types=[
            pltpu.VMEM((_RPW, _H), _F32),
            pltpu.VMEM((_RPW, _H), _F32),
            pltpu.VMEM((_RPW, _H), _F32),
            pltpu.VMEM((_RPW, 16), _F32),
            pltpu.VMEM((1, _H), _F32),
            pltpu.VMEM((_BINS, _H), _F32),
            pltpu.VMEM((_BINS, _H), _F32),
            pltpu.VMEM((_BINS, 16), _F32),
            pltpu.VMEM((_RPW,), jnp.int32),
        ],
        compiler_params=pltpu.CompilerParams(use_tc_tiling_on_sc=False),
    )
    def body(acc_hbm, y_hbm, dis_hbm, b_hbm, batch_hbm, osum_hbm, omax_hbm,
             ocnt_hbm, a0_v, a1_v, y_v, dis_v, b_v, sum_v, max_v, cnt_v,
             batch_v):
        cid = lax.axis_index("c")
        sid = lax.axis_index("s")
        w = cid * _NS + sid
        r0 = w * _RPW
        pltpu.sync_copy(acc_hbm.at[pl.ds(r0, _RPW)], a0_v)
        pltpu.sync_copy(acc_hbm.at[pl.ds(_NP + r0, _RPW)], a1_v)
        pltpu.sync_copy(y_hbm.at[pl.ds(r0, _RPW)], y_v)
        pltpu.sync_copy(dis_hbm.at[pl.ds(r0, _RPW)], dis_v)
        pltpu.sync_copy(b_hbm, b_v)
        pltpu.sync_copy(batch_hbm.at[pl.ds(r0, _RPW)], batch_v)

        zeros = jnp.zeros((16,), _F32)

        def zstep(r, carry):
            for c in range(_H // 16):
                sum_v[r, pl.ds(c * 16, 16)] = zeros
                max_v[r, pl.ds(c * 16, 16)] = zeros
            cnt_v[r, pl.ds(0, 16)] = zeros
            return carry
        lax.fori_loop(0, _BINS, zstep, 0)

        ones = jnp.ones((16,), _F32)

        def step(g, carry):
            bv = batch_v[pl.ds(g * 16, 16)]
            for ri in range(16):
                r = g * 16 + ri
                b = bv[ri]
                dv = dis_v[r, pl.ds(0, 16)]
                plsc.addupdate(cnt_v.at[b, pl.ds(0, 16)], ones)
                for c in range(_H // 16):
                    sl = pl.ds(c * 16, 16)
                    h = (a0_v[r, sl] + a1_v[r, sl] + y_v[r, sl]) * dv \
                        + b_v[0, sl]
                    h = jnp.maximum(h, 0.0)
                    plsc.addupdate(sum_v.at[b, sl], h)
                    max_v[b, sl] = jnp.maximum(max_v[b, sl], h)
            return carry
        lax.fori_loop(0, _RPW // 16, step, 0)

        o0 = w * _BINS
        pltpu.sync_copy(sum_v, osum_hbm.at[pl.ds(o0, _BINS)])
        pltpu.sync_copy(max_v, omax_hbm.at[pl.ds(o0, _BINS)])
        pltpu.sync_copy(cnt_v, ocnt_hbm.at[pl.ds(o0, _BINS)])

    return body(acc2, y, dis16, b_row, batch_pad)


def _tc_prep1(deg2, x, W1):
    def body(deg_ref, x_ref, w_ref, y_ref, dis_ref):
        deg = deg_ref[0:_NP, 0:1] + deg_ref[_NP:2 * _NP, 0:1] + 1.0
        dis = lax.rsqrt(deg)
        xw = jnp.dot(x_ref[...], w_ref[...], preferred_element_type=_F32)
        y_ref[...] = xw * dis
        dis_ref[...] = jnp.broadcast_to(dis, (_NP, 16))
    return pl.pallas_call(
        body,
        out_shape=(jax.ShapeDtypeStruct((_NP, _H), _F32),
                   jax.ShapeDtypeStruct((_NP, 16), _F32)),
    )(deg2, x, W1)


def _tc_layer(acc2, y, dis16, b_row, Wn):
    def body(acc_ref, y_ref, dis_ref, b_ref, w_ref, o_ref):
        dis = dis_ref[:, 0:1]
        s = acc_ref[0:_NP, :] + acc_ref[_NP:2 * _NP, :] + y_ref[...]
        h = jnp.maximum(s * dis + b_ref[...], 0.0)
        o_ref[...] = jnp.dot(h, w_ref[...], preferred_element_type=_F32) \
            * dis
    return pl.pallas_call(
        body, out_shape=jax.ShapeDtypeStruct((_NP, _H), _F32),
    )(acc2, y, dis16, b_row, Wn)


def _tc_head(psum, pmax, pcnt, metadata, Wm, bm_row, Wp1, bp1_row,
             Wp2, bp2_row):
    def body(ps_ref, pm_ref, pc_ref, md_ref, wm_ref, bm_ref, wp1_ref,
             bp1_ref, wp2_ref, bp2_ref, o_ref):
        s = ps_ref[0:_G, :]
        mx = pm_ref[0:_G, :]
        c = pc_ref[0:_G, 0:1]
        for wi in range(1, _NW):
            o = wi * _BINS
            s = s + ps_ref[o:o + _G, :]
            mx = jnp.maximum(mx, pm_ref[o:o + _G, :])
            c = c + pc_ref[o:o + _G, 0:1]
        mean = s / jnp.maximum(c, 1.0)
        meta = jnp.maximum(
            jnp.dot(md_ref[...], wm_ref[...], preferred_element_type=_F32)
            + bm_ref[...], 0.0)
        fused = jnp.concatenate([mean, mx, meta], axis=1)
        hid = jnp.maximum(
            jnp.dot(fused, wp1_ref[...], preferred_element_type=_F32)
            + bp1_ref[...], 0.0)
        o_ref[...] = jnp.dot(hid, wp2_ref[...], preferred_element_type=_F32) \
            + bp2_ref[...]
    return pl.pallas_call(
        body,
        out_shape=jax.ShapeDtypeStruct((_G, 1), _F32),
    )(psum, pmax, pcnt, metadata, Wm, bm_row, Wp1, bp1_row, Wp2, bp2_row)


def kernel(x, edge_index, batch, metadata, W1, b1, W2, b2, W3, b3, Wm, bm,
           Wp1, bp1, Wp2, bp2):
    src = edge_index[0]
    dst = edge_index[1]
    pad_e = _EP - src.shape[0]
    pad_idx = jnp.full((pad_e,), _N, jnp.int32)
    src2d = jnp.concatenate([src, pad_idx]).reshape(_EP // _CHUNK, _CHUNK)
    dst2d = jnp.concatenate([dst, pad_idx]).reshape(_EP // _CHUNK, _CHUNK)
    x_pad = jnp.pad(x, ((0, _NP - _N), (0, 0)))
    batch_pad = jnp.concatenate(
        [batch, jnp.full((_NP - _N,), _G, jnp.int32)])
    ones16 = jnp.ones((_CHUNK, 16), _F32)
    zero16 = jnp.zeros((_CHUNK, 16), _F32)
    zero64 = jnp.zeros((_CHUNK, _H), _F32)

    deg2 = _sc_degree(dst2d, ones16, zero16)
    y1, dis = _tc_prep1(deg2, x_pad, W1)
    acc1 = _sc_edge_pass(y1, src2d, dst2d, zero64)
    y2 = _tc_layer(acc1, y1, dis, b1.reshape(1, _H), W2)
    acc2 = _sc_edge_pass(y2, src2d, dst2d, zero64)
    y3 = _tc_layer(acc2, y2, dis, b2.reshape(1, _H), W3)
    acc3 = _sc_edge_pass(y3, src2d, dst2d, zero64)
    psum, pmax, pcnt = _sc_pool(acc3, y3, dis, b3.reshape(1, _H), batch_pad)
    out = _tc_head(psum, pmax, pcnt, metadata, Wm,
                   bm.reshape(1, 32), Wp1, bp1.reshape(1, 64), Wp2,
                   bp2.reshape(1, 1))
    return out


# speedup vs baseline: 1.1003x; 1.1003x over previous
import functools

import jax
import jax.numpy as jnp
from jax import lax
from jax.experimental import pallas as pl
from jax.experimental.pallas import tpu as pltpu
from jax.experimental.pallas import tpu_sc as plsc

_N, _D, _H, _G, _M = 10000, 128, 64, 128, 32
_NC, _NS = 2, 16
_NW = _NC * _NS
_NP = 10240
_CHUNK = 128
_CPW = 80
_CP0 = 120
_CP1 = 2 * _CPW - _CP0
_CPX = max(_CP0, _CP1)
_EP = _NW * _CPW * _CHUNK
_RPT = _NP // _NS
_F32 = jnp.float32


def _sc_mesh():
    return plsc.VectorSubcoreMesh(
        core_axis_name="c", subcore_axis_name="s",
        num_cores=_NC, num_subcores=_NS)


def _sc_degree(dst2d, ones16, zero16):
    @functools.partial(
        pl.kernel,
        out_type=jax.ShapeDtypeStruct((_NC * _NP, 16), _F32),
        mesh=_sc_mesh(),
        scratch_types=[
            pltpu.VMEM((_CPW, _CHUNK), jnp.int32),
            pltpu.VMEM((_CHUNK, 16), _F32),
            pltpu.VMEM((_RPT, 16), _F32),
            pltpu.VMEM_SHARED((_NP, 16), _F32),
            pltpu.SemaphoreType.DMA,
        ],
        compiler_params=pltpu.CompilerParams(use_tc_tiling_on_sc=False),
    )
    def body(dst_hbm, ones_hbm, zero_hbm, out_hbm, dst_v, ones_v, tmp_v,
             deg_sh, sem):
        cid = lax.axis_index("c")
        sid = lax.axis_index("s")
        w = cid * _NS + sid
        pltpu.sync_copy(zero_hbm, ones_v)
        for k in range(_RPT // _CHUNK):
            pltpu.sync_copy(ones_v, deg_sh.at[pl.ds(sid * _RPT + k * _CHUNK,
                                                    _CHUNK)])
        pltpu.sync_copy(ones_hbm, ones_v)
        pltpu.sync_copy(dst_hbm.at[pl.ds(w * _CPW, _CPW)], dst_v)
        plsc.subcore_barrier()

        def step(j, carry):
            pltpu.sync_copy(ones_v, deg_sh.at[dst_v.at[j]], add=True)
            return carry
        lax.fori_loop(0, _CPW, step, 0)
        plsc.subcore_barrier()

        pltpu.sync_copy(deg_sh.at[pl.ds(sid * _RPT, _RPT)], tmp_v)
        pltpu.sync_copy(tmp_v, out_hbm.at[pl.ds(cid * _NP + sid * _RPT, _RPT)])

    return body(dst2d, ones16, zero16)


def _sc_edge_pass(y, src2d, dst2d, zero64):
    @functools.partial(
        pl.kernel,
        out_type=jax.ShapeDtypeStruct((_NC * _NP, _H), _F32),
        mesh=_sc_mesh(),
        scratch_types=[
            pltpu.VMEM((_CPX, _CHUNK), jnp.int32),
            pltpu.VMEM((_CPX, _CHUNK), jnp.int32),
            pltpu.VMEM((_CHUNK, _H), _F32),
            pltpu.VMEM((_CHUNK, _H), _F32),
            pltpu.VMEM((_CHUNK, _H), _F32),
            pltpu.VMEM_SHARED((_NP, _H), _F32),
            pltpu.SemaphoreType.DMA,
            pltpu.SemaphoreType.DMA,
        ],
        compiler_params=pltpu.CompilerParams(use_tc_tiling_on_sc=False),
    )
    def body(y_hbm, src_hbm, dst_hbm, zero_hbm, out_hbm, src_v, dst_v,
             rows_a, rows_b, tmp_v, acc_sh, sem_a, sem_b):
        cid = lax.axis_index("c")
        sid = lax.axis_index("s")
        pltpu.sync_copy(zero_hbm, tmp_v)
        for k in range(_RPT // _CHUNK):
            pltpu.sync_copy(tmp_v, acc_sh.at[pl.ds(sid * _RPT + k * _CHUNK,
                                                   _CHUNK)])
        n = jnp.where(cid == 0, _CP0, _CP1)
        base = cid * (_NS * _CP0) + sid * n
        pltpu.sync_copy(src_hbm.at[pl.ds(base, _CPX)], src_v)
        pltpu.sync_copy(dst_hbm.at[pl.ds(base, _CPX)], dst_v)
        plsc.subcore_barrier()

        pltpu.async_copy(y_hbm.at[src_v.at[0]], rows_a, sem_a)
        half = n // 2

        def step(i, carry):
            j0 = 2 * i
            j1 = 2 * i + 1
            pltpu.make_async_copy(y_hbm.at[src_v.at[j0]], rows_a,
                                  sem_a).wait()
            pltpu.async_copy(y_hbm.at[src_v.at[j1]], rows_b, sem_b)
            pltpu.sync_copy(rows_a, acc_sh.at[dst_v.at[j0]], add=True)

            @pl.when(i < half - 1)
            def _():
                pltpu.async_copy(y_hbm.at[src_v.at[j0 + 2]], rows_a, sem_a)
            pltpu.make_async_copy(y_hbm.at[src_v.at[j1]], rows_b,
                                  sem_b).wait()
            pltpu.sync_copy(rows_b, acc_sh.at[dst_v.at[j1]], add=True)
            return carry
        lax.fori_loop(0, half, step, 0)
        plsc.subcore_barrier()

        for k in range(_RPT // _CHUNK):
            r0 = sid * _RPT + k * _CHUNK
            pltpu.sync_copy(acc_sh.at[pl.ds(r0, _CHUNK)], tmp_v)
            pltpu.sync_copy(tmp_v, out_hbm.at[pl.ds(cid * _NP + r0, _CHUNK)])

    return body(y, src2d, dst2d, zero64)


_BINS = 136
_RPW = _NP // _NW


def _sc_pool(acc2, y, dis16, b_row, batch_pad):
    @functools.partial(
        pl.kernel,
        out_type=(jax.ShapeDtypeStruct((_NW * _BINS, _H), _F32),
                  jax.ShapeDtypeStruct((_NW * _BINS, _H), _F32),
                  jax.ShapeDtypeStruct((_NW * _BINS, 16), _F32)),
        mesh=_sc_mesh(),
        scratch_types=[
            pltpu.VMEM((_RPW, _H), _F32),
            pltpu.VMEM((_RPW, _H), _F32),
            pltpu.VMEM((_RPW, _H), _F32),
            pltpu.VMEM((_RPW, 16), _F32),
            pltpu.VMEM((1, _H), _F32),
            pltpu.VMEM((_BINS, _H), _F32),
            pltpu.VMEM((_BINS, _H), _F32),
            pltpu.VMEM((_BINS, 16), _F32),
            pltpu.VMEM((_RPW,), jnp.int32),
        ],
        compiler_params=pltpu.CompilerParams(use_tc_tiling_on_sc=False),
    )
    def body(acc_hbm, y_hbm, dis_hbm, b_hbm, batch_hbm, osum_hbm, omax_hbm,
             ocnt_hbm, a0_v, a1_v, y_v, dis_v, b_v, sum_v, max_v, cnt_v,
             batch_v):
        cid = lax.axis_index("c")
        sid = lax.axis_index("s")
        w = cid * _NS + sid
        r0 = w * _RPW
        pltpu.sync_copy(acc_hbm.at[pl.ds(r0, _RPW)], a0_v)
        pltpu.sync_copy(acc_hbm.at[pl.ds(_NP + r0, _RPW)], a1_v)
        pltpu.sync_copy(y_hbm.at[pl.ds(r0, _RPW)], y_v)
        pltpu.sync_copy(dis_hbm.at[pl.ds(r0, _RPW)], dis_v)
        pltpu.sync_copy(b_hbm, b_v)
        pltpu.sync_copy(batch_hbm.at[pl.ds(r0, _RPW)], batch_v)

        zeros = jnp.zeros((16,), _F32)

        def zstep(r, carry):
            for c in range(_H // 16):
                sum_v[r, pl.ds(c * 16, 16)] = zeros
                max_v[r, pl.ds(c * 16, 16)] = zeros
            cnt_v[r, pl.ds(0, 16)] = zeros
            return carry
        lax.fori_loop(0, _BINS, zstep, 0)

        ones = jnp.ones((16,), _F32)

        def step(g, carry):
            bv = batch_v[pl.ds(g * 16, 16)]
            for ri in range(16):
                r = g * 16 + ri
                b = bv[ri]
                dv = dis_v[r, pl.ds(0, 16)]
                plsc.addupdate(cnt_v.at[b, pl.ds(0, 16)], ones)
                for c in range(_H // 16):
                    sl = pl.ds(c * 16, 16)
                    h = (a0_v[r, sl] + a1_v[r, sl] + y_v[r, sl]) * dv \
                        + b_v[0, sl]
                    h = jnp.maximum(h, 0.0)
                    plsc.addupdate(sum_v.at[b, sl], h)
                    max_v[b, sl] = jnp.maximum(max_v[b, sl], h)
            return carry
        lax.fori_loop(0, _RPW // 16, step, 0)

        o0 = w * _BINS
        pltpu.sync_copy(sum_v, osum_hbm.at[pl.ds(o0, _BINS)])
        pltpu.sync_copy(max_v, omax_hbm.at[pl.ds(o0, _BINS)])
        pltpu.sync_copy(cnt_v, ocnt_hbm.at[pl.ds(o0, _BINS)])

    return body(acc2, y, dis16, b_row, batch_pad)


def _tc_prep1(deg2, x, W1):
    def body(deg_ref, x_ref, w_ref, y_ref, dis_ref):
        deg = deg_ref[0:_NP, 0:1] + deg_ref[_NP:2 * _NP, 0:1] + 1.0
        dis = lax.rsqrt(deg)
        xw = jnp.dot(x_ref[...], w_ref[...], preferred_element_type=_F32)
        y_ref[...] = xw * dis
        dis_ref[...] = jnp.broadcast_to(dis, (_NP, 16))
    return pl.pallas_call(
        body,
        out_shape=(jax.ShapeDtypeStruct((_NP, _H), _F32),
                   jax.ShapeDtypeStruct((_NP, 16), _F32)),
    )(deg2, x, W1)


def _tc_layer(acc2, y, dis16, b_row, Wn):
    def body(acc_ref, y_ref, dis_ref, b_ref, w_ref, o_ref):
        dis = dis_ref[:, 0:1]
        s = acc_ref[0:_NP, :] + acc_ref[_NP:2 * _NP, :] + y_ref[...]
        h = jnp.maximum(s * dis + b_ref[...], 0.0)
        o_ref[...] = jnp.dot(h, w_ref[...], preferred_element_type=_F32) \
            * dis
    return pl.pallas_call(
        body, out_shape=jax.ShapeDtypeStruct((_NP, _H), _F32),
    )(acc2, y, dis16, b_row, Wn)


def _tc_head(psum, pmax, pcnt, metadata, Wm, bm_row, Wp1, bp1_row,
             Wp2, bp2_row):
    def body(ps_ref, pm_ref, pc_ref, md_ref, wm_ref, bm_ref, wp1_ref,
             bp1_ref, wp2_ref, bp2_ref, o_ref):
        s = ps_ref[0:_G, :]
        mx = pm_ref[0:_G, :]
        c = pc_ref[0:_G, 0:1]
        for wi in range(1, _NW):
            o = wi * _BINS
            s = s + ps_ref[o:o + _G, :]
            mx = jnp.maximum(mx, pm_ref[o:o + _G, :])
            c = c + pc_ref[o:o + _G, 0:1]
        mean = s / jnp.maximum(c, 1.0)
        meta = jnp.maximum(
            jnp.dot(md_ref[...], wm_ref[...], preferred_element_type=_F32)
            + bm_ref[...], 0.0)
        fused = jnp.concatenate([mean, mx, meta], axis=1)
        hid = jnp.maximum(
            jnp.dot(fused, wp1_ref[...], preferred_element_type=_F32)
            + bp1_ref[...], 0.0)
        o_ref[...] = jnp.dot(hid, wp2_ref[...], preferred_element_type=_F32) \
            + bp2_ref[...]
    return pl.pallas_call(
        body,
        out_shape=jax.ShapeDtypeStruct((_G, 1), _F32),
    )(psum, pmax, pcnt, metadata, Wm, bm_row, Wp1, bp1_row, Wp2, bp2_row)


def kernel(x, edge_index, batch, metadata, W1, b1, W2, b2, W3, b3, Wm, bm,
           Wp1, bp1, Wp2, bp2):
    src = edge_index[0]
    dst = edge_index[1]
    pad_e = _EP - src.shape[0]
    pad_idx = jnp.full((pad_e,), _N, jnp.int32)
    src2d = jnp.concatenate([src, pad_idx]).reshape(_EP // _CHUNK, _CHUNK)
    dst2d = jnp.concatenate([dst, pad_idx]).reshape(_EP // _CHUNK, _CHUNK)
    x_pad = jnp.pad(x, ((0, _NP - _N), (0, 0)))
    batch_pad = jnp.concatenate(
        [batch, jnp.full((_NP - _N,), _G, jnp.int32)])
    ones16 = jnp.ones((_CHUNK, 16), _F32)
    zero16 = jnp.zeros((_CHUNK, 16), _F32)
    zero64 = jnp.zeros((_CHUNK, _H), _F32)

    deg2 = _sc_degree(dst2d, ones16, zero16)
    y1, dis = _tc_prep1(deg2, x_pad, W1)
    acc1 = _sc_edge_pass(y1, src2d, dst2d, zero64)
    y2 = _tc_layer(acc1, y1, dis, b1.reshape(1, _H), W2)
    acc2 = _sc_edge_pass(y2, src2d, dst2d, zero64)
    y3 = _tc_layer(acc2, y2, dis, b2.reshape(1, _H), W3)
    acc3 = _sc_edge_pass(y3, src2d, dst2d, zero64)
    psum, pmax, pcnt = _sc_pool(acc3, y3, dis, b3.reshape(1, _H), batch_pad)
    out = _tc_head(psum, pmax, pcnt, metadata, Wm,
                   bm.reshape(1, 32), Wp1, bp1.reshape(1, 64), Wp2,
                   bp2.reshape(1, 1))
    return out
